# baseline (speedup 1.0000x reference)
import jax
import jax.numpy as jnp
from jax.experimental import pallas as pl
from jax.experimental.pallas import tpu as pltpu


def _router_body(x_ref, wr_ref, idx_ref, w_ref):
    t = x_ref.shape[1]
    e = wr_ref.shape[1]
    xb = x_ref[0]
    ones = jnp.full((1, t), 1.0 / t, dtype=jnp.float32)
    pooled = jnp.dot(ones, xb, precision=jax.lax.Precision.HIGHEST)
    logits = jnp.dot(pooled, wr_ref[...], precision=jax.lax.Precision.HIGHEST)
    iota = jax.lax.broadcasted_iota(jnp.int32, (1, e), 1)
    m1 = jnp.max(logits, axis=1, keepdims=True)
    i1 = jnp.min(jnp.where(logits == m1, iota, e), axis=1, keepdims=True)
    masked = jnp.where(iota == i1, -jnp.inf, logits)
    m2 = jnp.max(masked, axis=1, keepdims=True)
    i2 = jnp.min(jnp.where(masked == m2, iota, e), axis=1, keepdims=True)
    e2 = jnp.exp(m2 - m1)
    denom = 1.0 + e2
    idx_ref[0, :, 0:1] = i1
    idx_ref[0, :, 1:2] = i2
    w_ref[0, :, 0:1] = 1.0 / denom
    w_ref[0, :, 1:2] = e2 / denom


def _ffn_body(idx_ref, w_ref, x_ref, w1_ref, b1_ref, w2_ref, b2_ref, out_ref):
    b = pl.program_id(0)
    k = pl.program_id(2)
    f = pl.program_id(3)
    w = w_ref[b, k]
    xb = x_ref[0]
    h = jnp.dot(xb, w1_ref[0], preferred_element_type=jnp.float32)
    h = h + b1_ref[0]
    a = 0.5 * h * (1.0 + jax.lax.erf(h * 0.7071067811865476))
    contrib = jnp.dot(
        a.astype(w2_ref.dtype), w2_ref[0], preferred_element_type=jnp.float32
    )

    @pl.when(jnp.logical_and(k == 0, f == 0))
    def _init():
        out_ref[0] = jnp.zeros_like(out_ref[0])

    @pl.when(f == 0)
    def _bias():
        out_ref[0] = out_ref[0] + w * b2_ref[0]

    out_ref[0] = out_ref[0] + w * contrib


def kernel(x, W1, b1, W2, b2, Wr):
    B, T, D = x.shape
    E, _, F = W1.shape
    K = 2
    T_t = 1024
    F_t = 1024

    idx3, wts3 = pl.pallas_call(
        _router_body,
        grid=(B,),
        in_specs=[
            pl.BlockSpec((1, T, D), lambda b: (b, 0, 0)),
            pl.BlockSpec((D, E), lambda b: (0, 0)),
        ],
        out_specs=[
            pl.BlockSpec((1, 1, K), lambda b: (b, 0, 0)),
            pl.BlockSpec((1, 1, K), lambda b: (b, 0, 0)),
        ],
        out_shape=[
            jax.ShapeDtypeStruct((B, 1, K), jnp.int32),
            jax.ShapeDtypeStruct((B, 1, K), jnp.float32),
        ],
    )(x, Wr)
    top_idx = idx3.reshape(B, K)
    wts = wts3.reshape(B, K)

    b1r = b1.reshape(E, 1, F)
    b2r = b2.reshape(E, 1, D)
    xc = x.astype(jnp.bfloat16)
    W1c = W1.astype(jnp.bfloat16)
    W2c = W2.astype(jnp.bfloat16)

    grid_spec = pltpu.PrefetchScalarGridSpec(
        num_scalar_prefetch=2,
        grid=(B, T // T_t, K, F // F_t),
        in_specs=[
            pl.BlockSpec((1, T_t, D), lambda b, t, k, f, ir, wr: (b, t, 0)),
            pl.BlockSpec((1, D, F_t), lambda b, t, k, f, ir, wr: (ir[b, k], 0, f)),
            pl.BlockSpec((1, 1, F_t), lambda b, t, k, f, ir, wr: (ir[b, k], 0, f)),
            pl.BlockSpec((1, F_t, D), lambda b, t, k, f, ir, wr: (ir[b, k], f, 0)),
            pl.BlockSpec((1, 1, D), lambda b, t, k, f, ir, wr: (ir[b, k], 0, 0)),
        ],
        out_specs=pl.BlockSpec((1, T_t, D), lambda b, t, k, f, ir, wr: (b, t, 0)),
    )
    out = pl.pallas_call(
        _ffn_body,
        grid_spec=grid_spec,
        out_shape=jax.ShapeDtypeStruct((B, T, D), jnp.float32),
        compiler_params=pltpu.CompilerParams(
            dimension_semantics=("parallel", "parallel", "arbitrary", "arbitrary"),
        ),
    )(top_idx, wts, xc, W1c, b1r, W2c, b2r)
    return out


# speedup vs baseline: 1.6401x; 1.6401x over previous
import jax
import jax.numpy as jnp
from jax.experimental import pallas as pl
from jax.experimental.pallas import tpu as pltpu


def _router_body(x_ref, wr_ref, idx_ref, w_ref):
    t = x_ref.shape[1]
    e = wr_ref.shape[1]
    xb = x_ref[0]
    ones = jnp.full((1, t), 1.0 / t, dtype=jnp.float32)
    pooled = jnp.dot(ones, xb, precision=jax.lax.Precision.HIGHEST)
    logits = jnp.dot(pooled, wr_ref[...], precision=jax.lax.Precision.HIGHEST)
    iota = jax.lax.broadcasted_iota(jnp.int32, (1, e), 1)
    m1 = jnp.max(logits, axis=1, keepdims=True)
    i1 = jnp.min(jnp.where(logits == m1, iota, e), axis=1, keepdims=True)
    masked = jnp.where(iota == i1, -jnp.inf, logits)
    m2 = jnp.max(masked, axis=1, keepdims=True)
    i2 = jnp.min(jnp.where(masked == m2, iota, e), axis=1, keepdims=True)
    e2 = jnp.exp(m2 - m1)
    denom = 1.0 + e2
    idx_ref[0, :, 0:1] = i1
    idx_ref[0, :, 1:2] = i2
    w_ref[0, :, 0:1] = 1.0 / denom
    w_ref[0, :, 1:2] = e2 / denom


def _ffn_body(idx_ref, w_ref, x_ref, w1_ref, b1_ref, w2_ref, b2_ref, out_ref):
    b = pl.program_id(0)
    k = pl.program_id(2)
    f = pl.program_id(3)
    w = w_ref[b, k]
    xb = x_ref[0].astype(jnp.bfloat16)
    h = jnp.dot(
        xb, w1_ref[0].astype(jnp.bfloat16), preferred_element_type=jnp.float32
    )
    h = h + b1_ref[0]
    a = 0.5 * h * (1.0 + jax.lax.erf(h * 0.7071067811865476))
    contrib = jnp.dot(
        a.astype(jnp.bfloat16),
        w2_ref[0].astype(jnp.bfloat16),
        preferred_element_type=jnp.float32,
    )

    @pl.when(jnp.logical_and(k == 0, f == 0))
    def _init():
        out_ref[0] = jnp.zeros_like(out_ref[0])

    @pl.when(f == 0)
    def _bias():
        out_ref[0] = out_ref[0] + w * b2_ref[0]

    out_ref[0] = out_ref[0] + w * contrib


def kernel(x, W1, b1, W2, b2, Wr):
    B, T, D = x.shape
    E, _, F = W1.shape
    K = 2
    T_t = 1024
    F_t = 1024

    idx3, wts3 = pl.pallas_call(
        _router_body,
        grid=(B,),
        in_specs=[
            pl.BlockSpec((1, T, D), lambda b: (b, 0, 0)),
            pl.BlockSpec((D, E), lambda b: (0, 0)),
        ],
        out_specs=[
            pl.BlockSpec((1, 1, K), lambda b: (b, 0, 0)),
            pl.BlockSpec((1, 1, K), lambda b: (b, 0, 0)),
        ],
        out_shape=[
            jax.ShapeDtypeStruct((B, 1, K), jnp.int32),
            jax.ShapeDtypeStruct((B, 1, K), jnp.float32),
        ],
    )(x, Wr)
    top_idx = idx3.reshape(B, K)
    wts = wts3.reshape(B, K)

    b1r = b1.reshape(E, 1, F)
    b2r = b2.reshape(E, 1, D)

    grid_spec = pltpu.PrefetchScalarGridSpec(
        num_scalar_prefetch=2,
        grid=(B, T // T_t, K, F // F_t),
        in_specs=[
            pl.BlockSpec((1, T_t, D), lambda b, t, k, f, ir, wr: (b, t, 0)),
            pl.BlockSpec((1, D, F_t), lambda b, t, k, f, ir, wr: (ir[b, k], 0, f)),
            pl.BlockSpec((1, 1, F_t), lambda b, t, k, f, ir, wr: (ir[b, k], 0, f)),
            pl.BlockSpec((1, F_t, D), lambda b, t, k, f, ir, wr: (ir[b, k], f, 0)),
            pl.BlockSpec((1, 1, D), lambda b, t, k, f, ir, wr: (ir[b, k], 0, 0)),
        ],
        out_specs=pl.BlockSpec((1, T_t, D), lambda b, t, k, f, ir, wr: (b, t, 0)),
    )
    out = pl.pallas_call(
        _ffn_body,
        grid_spec=grid_spec,
        out_shape=jax.ShapeDtypeStruct((B, T, D), jnp.float32),
        compiler_params=pltpu.CompilerParams(
            dimension_semantics=("parallel", "parallel", "arbitrary", "arbitrary"),
        ),
    )(top_idx, wts, x, W1, b1r, W2, b2r)
    return out


# speedup vs baseline: 1.6863x; 1.0282x over previous
import jax
import jax.numpy as jnp
from jax.experimental import pallas as pl
from jax.experimental.pallas import tpu as pltpu


def _router_body(x_ref, wr_ref, idx_ref, w_ref):
    t = x_ref.shape[1]
    e = wr_ref.shape[1]
    xb = x_ref[0]
    ones = jnp.full((1, t), 1.0 / t, dtype=jnp.float32)
    pooled = jnp.dot(ones, xb, precision=jax.lax.Precision.HIGHEST)
    logits = jnp.dot(pooled, wr_ref[...], precision=jax.lax.Precision.HIGHEST)
    iota = jax.lax.broadcasted_iota(jnp.int32, (1, e), 1)
    m1 = jnp.max(logits, axis=1, keepdims=True)
    i1 = jnp.min(jnp.where(logits == m1, iota, e), axis=1, keepdims=True)
    masked = jnp.where(iota == i1, -jnp.inf, logits)
    m2 = jnp.max(masked, axis=1, keepdims=True)
    i2 = jnp.min(jnp.where(masked == m2, iota, e), axis=1, keepdims=True)
    e2 = jnp.exp(m2 - m1)
    denom = 1.0 + e2
    idx_ref[0, :, 0:1] = i1
    idx_ref[0, :, 1:2] = i2
    w_ref[0, :, 0:1] = 1.0 / denom
    w_ref[0, :, 1:2] = e2 / denom


def _ffn_body(idx_ref, w_ref, x_ref, w1_ref, b1_ref, w2_ref, b2_ref, out_ref):
    b = pl.program_id(0)
    k = pl.program_id(2)
    f = pl.program_id(3)
    w = w_ref[b, k]
    xb = x_ref[0].astype(jnp.bfloat16)
    h = jnp.dot(
        xb, w1_ref[0].astype(jnp.bfloat16), preferred_element_type=jnp.float32
    )
    h = h + b1_ref[0]
    a = 0.5 * h * (1.0 + jax.lax.erf(h * 0.7071067811865476))
    contrib = jnp.dot(
        a.astype(jnp.bfloat16),
        w2_ref[0].astype(jnp.bfloat16),
        preferred_element_type=jnp.float32,
    )

    @pl.when(jnp.logical_and(k == 0, f == 0))
    def _init():
        out_ref[0] = jnp.zeros_like(out_ref[0])

    @pl.when(f == 0)
    def _bias():
        out_ref[0] = out_ref[0] + w * b2_ref[0]

    out_ref[0] = out_ref[0] + w * contrib


def kernel(x, W1, b1, W2, b2, Wr):
    B, T, D = x.shape
    E, _, F = W1.shape
    K = 2
    T_t = 2048
    F_t = 1024

    idx3, wts3 = pl.pallas_call(
        _router_body,
        grid=(B,),
        in_specs=[
            pl.BlockSpec((1, T, D), lambda b: (b, 0, 0)),
            pl.BlockSpec((D, E), lambda b: (0, 0)),
        ],
        out_specs=[
            pl.BlockSpec((1, 1, K), lambda b: (b, 0, 0)),
            pl.BlockSpec((1, 1, K), lambda b: (b, 0, 0)),
        ],
        out_shape=[
            jax.ShapeDtypeStruct((B, 1, K), jnp.int32),
            jax.ShapeDtypeStruct((B, 1, K), jnp.float32),
        ],
    )(x, Wr)
    top_idx = idx3.reshape(B, K)
    wts = wts3.reshape(B, K)

    b1r = b1.reshape(E, 1, F)
    b2r = b2.reshape(E, 1, D)

    grid_spec = pltpu.PrefetchScalarGridSpec(
        num_scalar_prefetch=2,
        grid=(B, T // T_t, K, F // F_t),
        in_specs=[
            pl.BlockSpec((1, T_t, D), lambda b, t, k, f, ir, wr: (b, t, 0)),
            pl.BlockSpec((1, D, F_t), lambda b, t, k, f, ir, wr: (ir[b, k], 0, f)),
            pl.BlockSpec((1, 1, F_t), lambda b, t, k, f, ir, wr: (ir[b, k], 0, f)),
            pl.BlockSpec((1, F_t, D), lambda b, t, k, f, ir, wr: (ir[b, k], f, 0)),
            pl.BlockSpec((1, 1, D), lambda b, t, k, f, ir, wr: (ir[b, k], 0, 0)),
        ],
        out_specs=pl.BlockSpec((1, T_t, D), lambda b, t, k, f, ir, wr: (b, t, 0)),
    )
    out = pl.pallas_call(
        _ffn_body,
        grid_spec=grid_spec,
        out_shape=jax.ShapeDtypeStruct((B, T, D), jnp.float32),
        compiler_params=pltpu.CompilerParams(
            dimension_semantics=("parallel", "parallel", "arbitrary", "arbitrary"),
        ),
    )(top_idx, wts, x, W1, b1r, W2, b2r)
    return out
